# baseline (speedup 1.0000x reference)
import functools

import jax
import jax.numpy as jnp
from jax import lax
from jax.experimental import pallas as pl
from jax.experimental.pallas import tpu as pltpu
from jax.experimental.pallas import tpu_sc as plsc

NC = 2
NS = 16
L = 16
NW = NC * NS
B = 16384
D = 32
BPW = B // NW
NCHUNK = 4
CHUNK = BPW // NCHUNK


def _bpr_body(user_t, item_t, uidx, pidx, nidx, pos_out, neg_out,
              uidx_v, pidx_v, nidx_v, urows, prows, nrows, psc, nsc, sem):
    wid = lax.axis_index("s") * NC + lax.axis_index("c")
    base = wid * BPW

    pltpu.sync_copy(uidx.at[wid], uidx_v)
    pltpu.sync_copy(pidx.at[wid], pidx_v)
    pltpu.sync_copy(nidx.at[wid], nidx_v)

    handles = []
    for j in range(NCHUNK):
        handles.append(pltpu.async_copy(user_t.at[uidx_v.at[j]], urows.at[j], sem))
        handles.append(pltpu.async_copy(item_t.at[pidx_v.at[j]], prows.at[j], sem))
        handles.append(pltpu.async_copy(item_t.at[nidx_v.at[j]], nrows.at[j], sem))
    for h in handles:
        h.wait()

    lane = lax.iota(jnp.int32, L)
    zeros = jnp.zeros((L,), jnp.float32)
    for j in range(NCHUNK):
        def blk_body(b, carry, j=j):
            r0 = b * L
            pvec = zeros
            nvec = zeros
            for k in range(L):
                r = r0 + k
                u0 = urows[j, r, pl.ds(0, L)]
                u1 = urows[j, r, pl.ds(L, L)]
                p0 = prows[j, r, pl.ds(0, L)]
                p1 = prows[j, r, pl.ds(L, L)]
                n0 = nrows[j, r, pl.ds(0, L)]
                n1 = nrows[j, r, pl.ds(L, L)]
                pvec = jnp.where(lane == k, jnp.sum(u0 * p0 + u1 * p1), pvec)
                nvec = jnp.where(lane == k, jnp.sum(u0 * n0 + u1 * n1), nvec)
            psc[pl.ds(j * CHUNK + r0, L)] = pvec
            nsc[pl.ds(j * CHUNK + r0, L)] = nvec
            return carry
        lax.fori_loop(0, CHUNK // L, blk_body, 0)

    pltpu.sync_copy(psc, pos_out.at[pl.ds(base, BPW)])
    pltpu.sync_copy(nsc, neg_out.at[pl.ds(base, BPW)])


_bpr_call = functools.partial(
    pl.kernel,
    out_type=(
        jax.ShapeDtypeStruct((B,), jnp.float32),
        jax.ShapeDtypeStruct((B,), jnp.float32),
    ),
    mesh=plsc.VectorSubcoreMesh(core_axis_name="c", subcore_axis_name="s"),
    compiler_params=pltpu.CompilerParams(
        needs_layout_passes=False, use_tc_tiling_on_sc=False
    ),
    scratch_types=[
        pltpu.VMEM((NCHUNK, CHUNK), jnp.int32),
        pltpu.VMEM((NCHUNK, CHUNK), jnp.int32),
        pltpu.VMEM((NCHUNK, CHUNK), jnp.int32),
        pltpu.VMEM((NCHUNK, CHUNK, D), jnp.float32),
        pltpu.VMEM((NCHUNK, CHUNK, D), jnp.float32),
        pltpu.VMEM((NCHUNK, CHUNK, D), jnp.float32),
        pltpu.VMEM((BPW,), jnp.float32),
        pltpu.VMEM((BPW,), jnp.float32),
        pltpu.SemaphoreType.DMA,
    ],
)(_bpr_body)


@jax.jit
def kernel(user_table, item_table, user_inputs, pos_inputs, neg_inputs):
    uidx = user_inputs.reshape(NW, NCHUNK, CHUNK).astype(jnp.int32)
    pidx = pos_inputs.reshape(NW, NCHUNK, CHUNK).astype(jnp.int32)
    nidx = neg_inputs.reshape(NW, NCHUNK, CHUNK).astype(jnp.int32)
    pos, neg = _bpr_call(user_table, item_table, uidx, pidx, nidx)
    return pos.reshape(B, 1), neg.reshape(B, 1)
